# baseline (speedup 1.0000x reference)
import functools

import jax
import jax.numpy as jnp
from jax import lax
from jax.experimental import pallas as pl
from jax.experimental.pallas import tpu as pltpu
from jax.experimental.pallas import tpu_sc as plsc

N_NODES = 10000
MAX_DEG = 32
D_FEAT = 128
HIDDEN = 512
S_HOP2 = 25
S_HOP1 = 10
BATCH = 512
NW = 32


def _make_sc_gather(V, D, B, dtype, chunk):
    per_w = B // NW
    assert B % NW == 0 and per_w % chunk == 0 and chunk % 8 == 0 and chunk <= 128
    n_chunks = per_w // chunk
    pipelined = n_chunks % 4 == 0
    mesh = plsc.VectorSubcoreMesh(core_axis_name="c", subcore_axis_name="s")

    @functools.partial(
        pl.kernel,
        mesh=mesh,
        out_type=jax.ShapeDtypeStruct((B, D), dtype),
        scratch_types=[
            pltpu.VMEM((per_w,), jnp.int32),
        ] + [pltpu.VMEM((chunk, D), dtype) for _ in range(4 if pipelined else 1)]
          + [pltpu.SemaphoreType.DMA for _ in range(8 if pipelined else 1)],
    )
    def gk(table_hbm, idx_hbm, out_hbm, idx_v, *bufs_sems):
        wid = lax.axis_index("s") * 2 + lax.axis_index("c")
        base = wid * per_w
        pltpu.sync_copy(idx_hbm.at[pl.ds(base, per_w)], idx_v)

        if not pipelined:
            buf_v, sem = bufs_sems

            def body(c, carry):
                pltpu.async_copy(table_hbm.at[idx_v.at[pl.ds(c * chunk, chunk)]],
                                 buf_v, sem).wait()
                pltpu.sync_copy(buf_v, out_hbm.at[pl.ds(base + c * chunk, chunk)])
                return carry

            lax.fori_loop(0, n_chunks, body, 0)
            return

        bufs = bufs_sems[:4]
        gsems = bufs_sems[4:8]
        wsems = bufs_sems[8:12]
        outer = n_chunks // 4

        def gstart(c, b):
            pltpu.async_copy(table_hbm.at[idx_v.at[pl.ds(c * chunk, chunk)]],
                             bufs[b], gsems[b])

        def gwait(c, b):
            pltpu.make_async_copy(
                table_hbm.at[idx_v.at[pl.ds(c * chunk, chunk)]],
                bufs[b], gsems[b]).wait()

        def wstart(c, b):
            pltpu.async_copy(bufs[b], out_hbm.at[pl.ds(base + c * chunk, chunk)],
                             wsems[b])

        def wwait(c, b):
            pltpu.make_async_copy(
                bufs[b], out_hbm.at[pl.ds(base + c * chunk, chunk)],
                wsems[b]).wait()

        gstart(0, 0)
        gstart(1, 1)

        def body(o, carry):
            c0 = 4 * o

            @pl.when(o > 0)
            def _():
                wwait(c0 - 2, 2)
                wwait(c0 - 1, 3)

            gstart(c0 + 2, 2)
            gstart(c0 + 3, 3)

            gwait(c0, 0)
            wstart(c0, 0)
            gwait(c0 + 1, 1)
            wstart(c0 + 1, 1)

            @pl.when(o + 1 < outer)
            def _():
                wwait(c0, 0)
                wwait(c0 + 1, 1)
                gstart(c0 + 4, 0)
                gstart(c0 + 5, 1)

            gwait(c0 + 2, 2)
            wstart(c0 + 2, 2)
            gwait(c0 + 3, 3)
            wstart(c0 + 3, 3)
            return carry

        lax.fori_loop(0, outer, body, 0)
        last = n_chunks - 4
        wwait(last, 0)
        wwait(last + 1, 1)
        wwait(last + 2, 2)
        wwait(last + 3, 3)

    return gk


B_FEAT = 262144
_gather_feat = _make_sc_gather(N_NODES, D_FEAT, B_FEAT, jnp.float32, 128)
_gather_adj = _make_sc_gather(N_NODES, 128, 2 * BATCH, jnp.int32, 32)
_gather_m1 = _make_sc_gather(N_NODES, HIDDEN, 2 * BATCH * S_HOP1, jnp.float32, 40)
_gather_h10 = _make_sc_gather(N_NODES, 2 * D_FEAT, 2 * BATCH, jnp.float32, 32)

NB = 200


def _fused0_body(*refs):
    g_refs = refs[:S_HOP2]
    f_ref, w0_ref, b0_ref, nw0_ref, sw0_ref, w1_ref, b1_ref = refs[S_HOP2:S_HOP2 + 7]
    h10_ref, m1_ref = refs[S_HOP2 + 7:]
    w0 = w0_ref[...]
    m10 = None
    for j in range(S_HOP2):
        d = jnp.dot(g_refs[j][...], w0, preferred_element_type=jnp.float32)
        m25 = d if j == 0 else jnp.maximum(m25, d)
        if j == S_HOP1 - 1:
            m10 = m25
    b0 = b0_ref[...]
    p25 = jax.nn.relu(m25 + b0)
    p10 = jax.nn.relu(m10 + b0)
    s0 = jnp.dot(f_ref[...], sw0_ref[...], preferred_element_type=jnp.float32)
    n25 = jnp.dot(p25, nw0_ref[...], preferred_element_type=jnp.float32)
    n10 = jnp.dot(p10, nw0_ref[...], preferred_element_type=jnp.float32)
    h25 = jax.nn.relu(jnp.concatenate([s0, n25], axis=1))
    h10_ref[...] = jax.nn.relu(jnp.concatenate([s0, n10], axis=1))
    m1_ref[...] = jax.nn.relu(
        jnp.dot(h25, w1_ref[...], preferred_element_type=jnp.float32) + b1_ref[...])


RB = 256


def _final_body(m_ref, h_ref, sw1_ref, nw1_ref, o_ref):
    pooled = m_ref[0]
    for j in range(1, S_HOP1):
        pooled = jnp.maximum(pooled, m_ref[j])
    a = jnp.dot(h_ref[...], sw1_ref[...], preferred_element_type=jnp.float32)
    b = jnp.dot(pooled, nw1_ref[...], preferred_element_type=jnp.float32)
    o = jnp.concatenate([a, b], axis=1)
    s = jnp.sum(o * o, axis=1, keepdims=True)
    o_ref[...] = o * lax.rsqrt(jnp.maximum(s, 1e-12))


def kernel(batch1, batch2, features, adj, mlp_W0, mlp_b0, neigh_W0, self_W0,
           mlp_W1, mlp_b1, neigh_W1, self_W1):
    idx1 = jnp.pad(adj[:, :S_HOP2].T.reshape(-1), (0, B_FEAT - S_HOP2 * N_NODES))
    g25 = _gather_feat(features, idx1)

    slab_blocks = N_NODES // NB
    h10_tab, m1_tab = pl.pallas_call(
        _fused0_body,
        grid=(N_NODES // NB,),
        in_specs=[
            pl.BlockSpec((NB, D_FEAT), lambda i, k=k: (slab_blocks * k + i, 0))
            for k in range(S_HOP2)
        ] + [
            pl.BlockSpec((NB, D_FEAT), lambda i: (i, 0)),
            pl.BlockSpec((D_FEAT, HIDDEN), lambda i: (0, 0)),
            pl.BlockSpec((1, HIDDEN), lambda i: (0, 0)),
            pl.BlockSpec((HIDDEN, D_FEAT), lambda i: (0, 0)),
            pl.BlockSpec((D_FEAT, D_FEAT), lambda i: (0, 0)),
            pl.BlockSpec((2 * D_FEAT, HIDDEN), lambda i: (0, 0)),
            pl.BlockSpec((1, HIDDEN), lambda i: (0, 0)),
        ],
        out_specs=[
            pl.BlockSpec((NB, 2 * D_FEAT), lambda i: (i, 0)),
            pl.BlockSpec((NB, HIDDEN), lambda i: (i, 0)),
        ],
        out_shape=[
            jax.ShapeDtypeStruct((N_NODES, 2 * D_FEAT), jnp.float32),
            jax.ShapeDtypeStruct((N_NODES, HIDDEN), jnp.float32),
        ],
    )(*([g25] * S_HOP2), features, mlp_W0, mlp_b0.reshape(1, HIDDEN),
      neigh_W0, self_W0, mlp_W1, mlp_b1.reshape(1, HIDDEN))

    batch = jnp.concatenate([batch1, batch2])
    adj128 = jnp.pad(adj, ((0, 0), (0, 128 - MAX_DEG)))
    adjb = _gather_adj(adj128, batch)
    idx3 = adjb[:, :S_HOP1].T.reshape(-1)
    mrows = _gather_m1(m1_tab, idx3).reshape(S_HOP1, 2 * BATCH, HIDDEN)
    h10b = _gather_h10(h10_tab, batch)

    out = pl.pallas_call(
        _final_body,
        grid=(2 * BATCH // RB,),
        in_specs=[
            pl.BlockSpec((S_HOP1, RB, HIDDEN), lambda i: (0, i, 0)),
            pl.BlockSpec((RB, 2 * D_FEAT), lambda i: (i, 0)),
            pl.BlockSpec((2 * D_FEAT, D_FEAT), lambda i: (0, 0)),
            pl.BlockSpec((HIDDEN, D_FEAT), lambda i: (0, 0)),
        ],
        out_specs=pl.BlockSpec((RB, 2 * D_FEAT), lambda i: (i, 0)),
        out_shape=jax.ShapeDtypeStruct((2 * BATCH, 2 * D_FEAT), jnp.float32),
    )(mrows, h10b, self_W1, neigh_W1)

    return (out[:BATCH], out[BATCH:])


# speedup vs baseline: 1.0661x; 1.0661x over previous
import functools

import jax
import jax.numpy as jnp
from jax import lax
from jax.experimental import pallas as pl
from jax.experimental.pallas import tpu as pltpu
from jax.experimental.pallas import tpu_sc as plsc

N_NODES = 10000
MAX_DEG = 32
D_FEAT = 128
HIDDEN = 512
S_HOP2 = 25
S_HOP1 = 10
BATCH = 512
NW = 32


def _make_sc_gather(V, D, B, dtype, chunk):
    per_w = B // NW
    assert B % NW == 0 and per_w % chunk == 0 and chunk % 8 == 0 and chunk <= 128
    n_chunks = per_w // chunk
    idx2d = n_chunks % 8 == 0
    pipelined = idx2d and n_chunks % 4 == 0
    mesh = plsc.VectorSubcoreMesh(core_axis_name="c", subcore_axis_name="s")

    @functools.partial(
        pl.kernel,
        mesh=mesh,
        out_type=jax.ShapeDtypeStruct((B, D), dtype),
        scratch_types=[
            pltpu.VMEM((n_chunks, chunk) if idx2d else (per_w,), jnp.int32),
        ] + [pltpu.VMEM((chunk, D), dtype) for _ in range(4 if pipelined else 1)]
          + [pltpu.SemaphoreType.DMA for _ in range(8 if pipelined else 1)],
    )
    def gk(table_hbm, idx_hbm, out_hbm, idx_v, *bufs_sems):
        wid = lax.axis_index("s") * 2 + lax.axis_index("c")
        base = wid * per_w

        def idx_at(c):
            return idx_v.at[c] if idx2d else idx_v.at[pl.ds(c * chunk, chunk)]

        if idx2d:
            pltpu.sync_copy(idx_hbm.at[pl.ds(wid * n_chunks, n_chunks)], idx_v)
        else:
            pltpu.sync_copy(idx_hbm.at[pl.ds(base, per_w)], idx_v)

        if not pipelined:
            buf_v, sem = bufs_sems

            def body(c, carry):
                pltpu.async_copy(table_hbm.at[idx_at(c)], buf_v, sem).wait()
                pltpu.sync_copy(buf_v, out_hbm.at[pl.ds(base + c * chunk, chunk)])
                return carry

            lax.fori_loop(0, n_chunks, body, 0)
            return

        bufs = bufs_sems[:4]
        gsems = bufs_sems[4:8]
        wsems = bufs_sems[8:12]
        outer = n_chunks // 4

        def gstart(c, b):
            pltpu.async_copy(table_hbm.at[idx_at(c)], bufs[b], gsems[b])

        def gwait(c, b):
            pltpu.make_async_copy(
                table_hbm.at[idx_at(c)], bufs[b], gsems[b]).wait()

        def wstart(c, b):
            pltpu.async_copy(bufs[b], out_hbm.at[pl.ds(base + c * chunk, chunk)],
                             wsems[b])

        def wwait(c, b):
            pltpu.make_async_copy(
                bufs[b], out_hbm.at[pl.ds(base + c * chunk, chunk)],
                wsems[b]).wait()

        gstart(0, 0)
        gstart(1, 1)

        def body(o, carry):
            c0 = 4 * o

            @pl.when(o > 0)
            def _():
                wwait(c0 - 2, 2)
                wwait(c0 - 1, 3)

            gstart(c0 + 2, 2)
            gstart(c0 + 3, 3)

            gwait(c0, 0)
            wstart(c0, 0)
            gwait(c0 + 1, 1)
            wstart(c0 + 1, 1)

            @pl.when(o + 1 < outer)
            def _():
                wwait(c0, 0)
                wwait(c0 + 1, 1)
                gstart(c0 + 4, 0)
                gstart(c0 + 5, 1)

            gwait(c0 + 2, 2)
            wstart(c0 + 2, 2)
            gwait(c0 + 3, 3)
            wstart(c0 + 3, 3)
            return carry

        lax.fori_loop(0, outer, body, 0)
        last = n_chunks - 4
        wwait(last, 0)
        wwait(last + 1, 1)
        wwait(last + 2, 2)
        wwait(last + 3, 3)

    def call(table, idx_flat):
        if idx2d:
            return gk(table, idx_flat.reshape(NW * n_chunks, chunk))
        return gk(table, idx_flat)

    return call


B_FEAT = 262144
_gather_feat = _make_sc_gather(N_NODES, D_FEAT, B_FEAT, jnp.float32, 128)
_gather_adj = _make_sc_gather(N_NODES, 128, 2 * BATCH, jnp.int32, 32)
_gather_m1 = _make_sc_gather(N_NODES, HIDDEN, 2 * BATCH * S_HOP1, jnp.float32, 40)
_gather_h10 = _make_sc_gather(N_NODES, 2 * D_FEAT, 2 * BATCH, jnp.float32, 32)

NB = 200


def _fused0_body(*refs):
    g_refs = refs[:S_HOP2]
    f_ref, w0_ref, b0_ref, nw0_ref, sw0_ref, w1_ref, b1_ref = refs[S_HOP2:S_HOP2 + 7]
    h10_ref, m1_ref = refs[S_HOP2 + 7:]
    w0 = w0_ref[...]
    m10 = None
    for j in range(S_HOP2):
        d = jnp.dot(g_refs[j][...], w0, preferred_element_type=jnp.float32)
        m25 = d if j == 0 else jnp.maximum(m25, d)
        if j == S_HOP1 - 1:
            m10 = m25
    b0 = b0_ref[...]
    p25 = jax.nn.relu(m25 + b0)
    p10 = jax.nn.relu(m10 + b0)
    s0 = jnp.dot(f_ref[...], sw0_ref[...], preferred_element_type=jnp.float32)
    n25 = jnp.dot(p25, nw0_ref[...], preferred_element_type=jnp.float32)
    n10 = jnp.dot(p10, nw0_ref[...], preferred_element_type=jnp.float32)
    h25 = jax.nn.relu(jnp.concatenate([s0, n25], axis=1))
    h10_ref[...] = jax.nn.relu(jnp.concatenate([s0, n10], axis=1))
    m1_ref[...] = jax.nn.relu(
        jnp.dot(h25, w1_ref[...], preferred_element_type=jnp.float32) + b1_ref[...])


RB = 256


def _final_body(m_ref, h_ref, sw1_ref, nw1_ref, o_ref):
    pooled = m_ref[0]
    for j in range(1, S_HOP1):
        pooled = jnp.maximum(pooled, m_ref[j])
    a = jnp.dot(h_ref[...], sw1_ref[...], preferred_element_type=jnp.float32)
    b = jnp.dot(pooled, nw1_ref[...], preferred_element_type=jnp.float32)
    o = jnp.concatenate([a, b], axis=1)
    s = jnp.sum(o * o, axis=1, keepdims=True)
    o_ref[...] = o * lax.rsqrt(jnp.maximum(s, 1e-12))


def kernel(batch1, batch2, features, adj, mlp_W0, mlp_b0, neigh_W0, self_W0,
           mlp_W1, mlp_b1, neigh_W1, self_W1):
    idx1 = jnp.pad(adj[:, :S_HOP2].T.reshape(-1), (0, B_FEAT - S_HOP2 * N_NODES))
    g25 = _gather_feat(features, idx1)

    slab_blocks = N_NODES // NB
    h10_tab, m1_tab = pl.pallas_call(
        _fused0_body,
        grid=(N_NODES // NB,),
        in_specs=[
            pl.BlockSpec((NB, D_FEAT), lambda i, k=k: (slab_blocks * k + i, 0))
            for k in range(S_HOP2)
        ] + [
            pl.BlockSpec((NB, D_FEAT), lambda i: (i, 0)),
            pl.BlockSpec((D_FEAT, HIDDEN), lambda i: (0, 0)),
            pl.BlockSpec((1, HIDDEN), lambda i: (0, 0)),
            pl.BlockSpec((HIDDEN, D_FEAT), lambda i: (0, 0)),
            pl.BlockSpec((D_FEAT, D_FEAT), lambda i: (0, 0)),
            pl.BlockSpec((2 * D_FEAT, HIDDEN), lambda i: (0, 0)),
            pl.BlockSpec((1, HIDDEN), lambda i: (0, 0)),
        ],
        out_specs=[
            pl.BlockSpec((NB, 2 * D_FEAT), lambda i: (i, 0)),
            pl.BlockSpec((NB, HIDDEN), lambda i: (i, 0)),
        ],
        out_shape=[
            jax.ShapeDtypeStruct((N_NODES, 2 * D_FEAT), jnp.float32),
            jax.ShapeDtypeStruct((N_NODES, HIDDEN), jnp.float32),
        ],
    )(*([g25] * S_HOP2), features, mlp_W0, mlp_b0.reshape(1, HIDDEN),
      neigh_W0, self_W0, mlp_W1, mlp_b1.reshape(1, HIDDEN))

    batch = jnp.concatenate([batch1, batch2])
    adj128 = jnp.pad(adj, ((0, 0), (0, 128 - MAX_DEG)))
    adjb = _gather_adj(adj128, batch)
    idx3 = adjb[:, :S_HOP1].T.reshape(-1)
    mrows = _gather_m1(m1_tab, idx3).reshape(S_HOP1, 2 * BATCH, HIDDEN)
    h10b = _gather_h10(h10_tab, batch)

    out = pl.pallas_call(
        _final_body,
        grid=(2 * BATCH // RB,),
        in_specs=[
            pl.BlockSpec((S_HOP1, RB, HIDDEN), lambda i: (0, i, 0)),
            pl.BlockSpec((RB, 2 * D_FEAT), lambda i: (i, 0)),
            pl.BlockSpec((2 * D_FEAT, D_FEAT), lambda i: (0, 0)),
            pl.BlockSpec((HIDDEN, D_FEAT), lambda i: (0, 0)),
        ],
        out_specs=pl.BlockSpec((RB, 2 * D_FEAT), lambda i: (i, 0)),
        out_shape=jax.ShapeDtypeStruct((2 * BATCH, 2 * D_FEAT), jnp.float32),
    )(mrows, h10b, self_W1, neigh_W1)

    return (out[:BATCH], out[BATCH:])


# speedup vs baseline: 1.3471x; 1.2635x over previous
import functools

import jax
import jax.numpy as jnp
from jax import lax
from jax.experimental import pallas as pl
from jax.experimental.pallas import tpu as pltpu
from jax.experimental.pallas import tpu_sc as plsc

N_NODES = 10000
MAX_DEG = 32
D_FEAT = 128
HIDDEN = 512
S_HOP2 = 25
S_HOP1 = 10
BATCH = 512
NW = 32


def _make_sc_gather(V, D, B, dtype, chunk):
    per_w = B // NW
    assert B % NW == 0 and per_w % chunk == 0 and chunk % 8 == 0 and chunk <= 128
    n_chunks = per_w // chunk
    idx2d = n_chunks % 8 == 0
    pipelined = idx2d and n_chunks % 4 == 0
    mesh = plsc.VectorSubcoreMesh(core_axis_name="c", subcore_axis_name="s")

    @functools.partial(
        pl.kernel,
        mesh=mesh,
        out_type=jax.ShapeDtypeStruct((B, D), dtype),
        scratch_types=[
            pltpu.VMEM((n_chunks, chunk) if idx2d else (per_w,), jnp.int32),
        ] + [pltpu.VMEM((chunk, D), dtype) for _ in range(4 if pipelined else 1)]
          + [pltpu.SemaphoreType.DMA for _ in range(8 if pipelined else 1)],
    )
    def gk(table_hbm, idx_hbm, out_hbm, idx_v, *bufs_sems):
        wid = lax.axis_index("s") * 2 + lax.axis_index("c")
        base = wid * per_w

        def idx_at(c):
            return idx_v.at[c] if idx2d else idx_v.at[pl.ds(c * chunk, chunk)]

        if idx2d:
            pltpu.sync_copy(idx_hbm.at[pl.ds(wid * n_chunks, n_chunks)], idx_v)
        else:
            pltpu.sync_copy(idx_hbm.at[pl.ds(base, per_w)], idx_v)

        if not pipelined:
            buf_v, sem = bufs_sems

            def body(c, carry):
                pltpu.async_copy(table_hbm.at[idx_at(c)], buf_v, sem).wait()
                pltpu.sync_copy(buf_v, out_hbm.at[pl.ds(base + c * chunk, chunk)])
                return carry

            lax.fori_loop(0, n_chunks, body, 0)
            return

        bufs = bufs_sems[:4]
        gsems = bufs_sems[4:8]
        wsems = bufs_sems[8:12]
        outer = n_chunks // 4

        def gstart(c, b):
            pltpu.async_copy(table_hbm.at[idx_at(c)], bufs[b], gsems[b])

        def gwait(c, b):
            pltpu.make_async_copy(
                table_hbm.at[idx_at(c)], bufs[b], gsems[b]).wait()

        def wstart(c, b):
            pltpu.async_copy(bufs[b], out_hbm.at[pl.ds(base + c * chunk, chunk)],
                             wsems[b])

        def wwait(c, b):
            pltpu.make_async_copy(
                bufs[b], out_hbm.at[pl.ds(base + c * chunk, chunk)],
                wsems[b]).wait()

        gstart(0, 0)
        gstart(1, 1)

        def body(o, carry):
            c0 = 4 * o

            @pl.when(o > 0)
            def _():
                wwait(c0 - 2, 2)
                wwait(c0 - 1, 3)

            gstart(c0 + 2, 2)
            gstart(c0 + 3, 3)

            gwait(c0, 0)
            wstart(c0, 0)
            gwait(c0 + 1, 1)
            wstart(c0 + 1, 1)

            @pl.when(o + 1 < outer)
            def _():
                wwait(c0, 0)
                wwait(c0 + 1, 1)
                gstart(c0 + 4, 0)
                gstart(c0 + 5, 1)

            gwait(c0 + 2, 2)
            wstart(c0 + 2, 2)
            gwait(c0 + 3, 3)
            wstart(c0 + 3, 3)
            return carry

        lax.fori_loop(0, outer, body, 0)
        last = n_chunks - 4
        wwait(last, 0)
        wwait(last + 1, 1)
        wwait(last + 2, 2)
        wwait(last + 3, 3)

    def call(table, idx_flat):
        if idx2d:
            return gk(table, idx_flat.reshape(NW * n_chunks, chunk))
        return gk(table, idx_flat)

    return call


KPAD = 10240
B_FEAT = S_HOP2 * KPAD
_gather_feat = _make_sc_gather(N_NODES, D_FEAT, B_FEAT, jnp.float32, 80)
_gather_adj = _make_sc_gather(N_NODES, 128, 2 * BATCH, jnp.int32, 32)
_gather_m1 = _make_sc_gather(N_NODES, HIDDEN, 2 * BATCH * S_HOP1, jnp.float32, 40)
_gather_h10 = _make_sc_gather(N_NODES, 2 * D_FEAT, 2 * BATCH, jnp.float32, 32)

NB = 200


def _fused0_body(g_ref, f_ref, w0_ref, b0_ref, nw0_ref, sw0_ref, w1_ref, b1_ref,
                 h10_ref, m1_ref):
    w0 = w0_ref[...]
    m10 = None
    for j in range(S_HOP2):
        d = jnp.dot(g_ref[j], w0, preferred_element_type=jnp.float32)
        m25 = d if j == 0 else jnp.maximum(m25, d)
        if j == S_HOP1 - 1:
            m10 = m25
    b0 = b0_ref[...]
    p25 = jax.nn.relu(m25 + b0)
    p10 = jax.nn.relu(m10 + b0)
    s0 = jnp.dot(f_ref[...], sw0_ref[...], preferred_element_type=jnp.float32)
    n25 = jnp.dot(p25, nw0_ref[...], preferred_element_type=jnp.float32)
    n10 = jnp.dot(p10, nw0_ref[...], preferred_element_type=jnp.float32)
    h25 = jax.nn.relu(jnp.concatenate([s0, n25], axis=1))
    h10_ref[...] = jax.nn.relu(jnp.concatenate([s0, n10], axis=1))
    m1_ref[...] = jax.nn.relu(
        jnp.dot(h25, w1_ref[...], preferred_element_type=jnp.float32) + b1_ref[...])


RB = 256


def _final_body(m_ref, h_ref, sw1_ref, nw1_ref, o_ref):
    pooled = m_ref[0]
    for j in range(1, S_HOP1):
        pooled = jnp.maximum(pooled, m_ref[j])
    a = jnp.dot(h_ref[...], sw1_ref[...], preferred_element_type=jnp.float32)
    b = jnp.dot(pooled, nw1_ref[...], preferred_element_type=jnp.float32)
    o = jnp.concatenate([a, b], axis=1)
    s = jnp.sum(o * o, axis=1, keepdims=True)
    o_ref[...] = o * lax.rsqrt(jnp.maximum(s, 1e-12))


def kernel(batch1, batch2, features, adj, mlp_W0, mlp_b0, neigh_W0, self_W0,
           mlp_W1, mlp_b1, neigh_W1, self_W1):
    idx1 = jnp.pad(adj[:, :S_HOP2].T, ((0, 0), (0, KPAD - N_NODES))).reshape(-1)
    g25 = _gather_feat(features, idx1).reshape(S_HOP2, KPAD, D_FEAT)

    h10_tab, m1_tab = pl.pallas_call(
        _fused0_body,
        grid=(N_NODES // NB,),
        in_specs=[
            pl.BlockSpec((S_HOP2, NB, D_FEAT), lambda i: (0, i, 0)),
        ] + [
            pl.BlockSpec((NB, D_FEAT), lambda i: (i, 0)),
            pl.BlockSpec((D_FEAT, HIDDEN), lambda i: (0, 0)),
            pl.BlockSpec((1, HIDDEN), lambda i: (0, 0)),
            pl.BlockSpec((HIDDEN, D_FEAT), lambda i: (0, 0)),
            pl.BlockSpec((D_FEAT, D_FEAT), lambda i: (0, 0)),
            pl.BlockSpec((2 * D_FEAT, HIDDEN), lambda i: (0, 0)),
            pl.BlockSpec((1, HIDDEN), lambda i: (0, 0)),
        ],
        out_specs=[
            pl.BlockSpec((NB, 2 * D_FEAT), lambda i: (i, 0)),
            pl.BlockSpec((NB, HIDDEN), lambda i: (i, 0)),
        ],
        out_shape=[
            jax.ShapeDtypeStruct((N_NODES, 2 * D_FEAT), jnp.float32),
            jax.ShapeDtypeStruct((N_NODES, HIDDEN), jnp.float32),
        ],
    )(g25, features, mlp_W0, mlp_b0.reshape(1, HIDDEN),
      neigh_W0, self_W0, mlp_W1, mlp_b1.reshape(1, HIDDEN))

    batch = jnp.concatenate([batch1, batch2])
    adj128 = jnp.pad(adj, ((0, 0), (0, 128 - MAX_DEG)))
    adjb = _gather_adj(adj128, batch)
    idx3 = adjb[:, :S_HOP1].T.reshape(-1)
    mrows = _gather_m1(m1_tab, idx3).reshape(S_HOP1, 2 * BATCH, HIDDEN)
    h10b = _gather_h10(h10_tab, batch)

    out = pl.pallas_call(
        _final_body,
        grid=(2 * BATCH // RB,),
        in_specs=[
            pl.BlockSpec((S_HOP1, RB, HIDDEN), lambda i: (0, i, 0)),
            pl.BlockSpec((RB, 2 * D_FEAT), lambda i: (i, 0)),
            pl.BlockSpec((2 * D_FEAT, D_FEAT), lambda i: (0, 0)),
            pl.BlockSpec((HIDDEN, D_FEAT), lambda i: (0, 0)),
        ],
        out_specs=pl.BlockSpec((RB, 2 * D_FEAT), lambda i: (i, 0)),
        out_shape=jax.ShapeDtypeStruct((2 * BATCH, 2 * D_FEAT), jnp.float32),
    )(mrows, h10b, self_W1, neigh_W1)

    return (out[:BATCH], out[BATCH:])


# speedup vs baseline: 1.4411x; 1.0698x over previous
import functools

import jax
import jax.numpy as jnp
from jax import lax
from jax.experimental import pallas as pl
from jax.experimental.pallas import tpu as pltpu
from jax.experimental.pallas import tpu_sc as plsc

N_NODES = 10000
MAX_DEG = 32
D_FEAT = 128
HIDDEN = 512
S_HOP2 = 25
S_HOP1 = 10
BATCH = 512
NW = 32


def _make_sc_gather(V, D, B, dtype, chunk):
    per_w = B // NW
    assert B % NW == 0 and per_w % chunk == 0 and chunk % 8 == 0 and chunk <= 128
    n_chunks = per_w // chunk
    idx2d = n_chunks % 8 == 0
    pipelined = idx2d and n_chunks % 4 == 0
    mesh = plsc.VectorSubcoreMesh(core_axis_name="c", subcore_axis_name="s")

    @functools.partial(
        pl.kernel,
        mesh=mesh,
        out_type=jax.ShapeDtypeStruct((B, D), dtype),
        scratch_types=[
            pltpu.VMEM((n_chunks, chunk) if idx2d else (per_w,), jnp.int32),
        ] + [pltpu.VMEM((chunk, D), dtype) for _ in range(4 if pipelined else 1)]
          + [pltpu.SemaphoreType.DMA for _ in range(8 if pipelined else 1)],
    )
    def gk(table_hbm, idx_hbm, out_hbm, idx_v, *bufs_sems):
        wid = lax.axis_index("s") * 2 + lax.axis_index("c")
        base = wid * per_w

        def idx_at(c):
            return idx_v.at[c] if idx2d else idx_v.at[pl.ds(c * chunk, chunk)]

        if idx2d:
            pltpu.sync_copy(idx_hbm.at[pl.ds(wid * n_chunks, n_chunks)], idx_v)
        else:
            pltpu.sync_copy(idx_hbm.at[pl.ds(base, per_w)], idx_v)

        if not pipelined:
            buf_v, sem = bufs_sems

            def body(c, carry):
                pltpu.async_copy(table_hbm.at[idx_at(c)], buf_v, sem).wait()
                pltpu.sync_copy(buf_v, out_hbm.at[pl.ds(base + c * chunk, chunk)])
                return carry

            lax.fori_loop(0, n_chunks, body, 0)
            return

        bufs = bufs_sems[:4]
        gsems = bufs_sems[4:8]
        wsems = bufs_sems[8:12]
        outer = n_chunks // 4

        def gstart(c, b):
            pltpu.async_copy(table_hbm.at[idx_at(c)], bufs[b], gsems[b])

        def gwait(c, b):
            pltpu.make_async_copy(
                table_hbm.at[idx_at(c)], bufs[b], gsems[b]).wait()

        def wstart(c, b):
            pltpu.async_copy(bufs[b], out_hbm.at[pl.ds(base + c * chunk, chunk)],
                             wsems[b])

        def wwait(c, b):
            pltpu.make_async_copy(
                bufs[b], out_hbm.at[pl.ds(base + c * chunk, chunk)],
                wsems[b]).wait()

        gstart(0, 0)
        gstart(1, 1)

        def body(o, carry):
            c0 = 4 * o

            @pl.when(o > 0)
            def _():
                wwait(c0 - 2, 2)
                wwait(c0 - 1, 3)

            gstart(c0 + 2, 2)
            gstart(c0 + 3, 3)

            gwait(c0, 0)
            wstart(c0, 0)
            gwait(c0 + 1, 1)
            wstart(c0 + 1, 1)

            @pl.when(o + 1 < outer)
            def _():
                wwait(c0, 0)
                wwait(c0 + 1, 1)
                gstart(c0 + 4, 0)
                gstart(c0 + 5, 1)

            gwait(c0 + 2, 2)
            wstart(c0 + 2, 2)
            gwait(c0 + 3, 3)
            wstart(c0 + 3, 3)
            return carry

        lax.fori_loop(0, outer, body, 0)
        last = n_chunks - 4
        wwait(last, 0)
        wwait(last + 1, 1)
        wwait(last + 2, 2)
        wwait(last + 3, 3)

    def call(table, idx_flat):
        if idx2d:
            return gk(table, idx_flat.reshape(NW * n_chunks, chunk))
        return gk(table, idx_flat)

    return call


KPAD = 10240
NSEG = 4
SEG = KPAD // NSEG
B_SEG = S_HOP2 * SEG
_gather_feat = _make_sc_gather(N_NODES, D_FEAT, B_SEG, jnp.float32, 80)
_gather_adj = _make_sc_gather(N_NODES, 128, 2 * BATCH, jnp.int32, 32)
_gather_m1 = _make_sc_gather(N_NODES, HIDDEN, 2 * BATCH * S_HOP1, jnp.float32, 40)
_gather_h10 = _make_sc_gather(N_NODES, 2 * D_FEAT, 2 * BATCH, jnp.float32, 32)

NB = 256


def _fused0_body(g_ref, f_ref, w0_ref, b0_ref, nw0_ref, sw0_ref, w1_ref, b1_ref,
                 h10_ref, m1_ref):
    w0 = w0_ref[...]
    m10 = None
    for j in range(S_HOP2):
        d = jnp.dot(g_ref[j], w0, preferred_element_type=jnp.float32)
        m25 = d if j == 0 else jnp.maximum(m25, d)
        if j == S_HOP1 - 1:
            m10 = m25
    b0 = b0_ref[...]
    p25 = jax.nn.relu(m25 + b0)
    p10 = jax.nn.relu(m10 + b0)
    s0 = jnp.dot(f_ref[...], sw0_ref[...], preferred_element_type=jnp.float32)
    n25 = jnp.dot(p25, nw0_ref[...], preferred_element_type=jnp.float32)
    n10 = jnp.dot(p10, nw0_ref[...], preferred_element_type=jnp.float32)
    h25 = jax.nn.relu(jnp.concatenate([s0, n25], axis=1))
    h10_ref[...] = jax.nn.relu(jnp.concatenate([s0, n10], axis=1))
    m1_ref[...] = jax.nn.relu(
        jnp.dot(h25, w1_ref[...], preferred_element_type=jnp.float32) + b1_ref[...])


RB = 256


def _final_body(m_ref, h_ref, sw1_ref, nw1_ref, o_ref):
    pooled = m_ref[0]
    for j in range(1, S_HOP1):
        pooled = jnp.maximum(pooled, m_ref[j])
    a = jnp.dot(h_ref[...], sw1_ref[...], preferred_element_type=jnp.float32)
    b = jnp.dot(pooled, nw1_ref[...], preferred_element_type=jnp.float32)
    o = jnp.concatenate([a, b], axis=1)
    s = jnp.sum(o * o, axis=1, keepdims=True)
    o_ref[...] = o * lax.rsqrt(jnp.maximum(s, 1e-12))


def kernel(batch1, batch2, features, adj, mlp_W0, mlp_b0, neigh_W0, self_W0,
           mlp_W1, mlp_b1, neigh_W1, self_W1):
    adjT = jnp.pad(adj[:, :S_HOP2].T, ((0, 0), (0, KPAD - N_NODES)))
    feat_pad = jnp.pad(features, ((0, KPAD - N_NODES), (0, 0)))

    h10_parts, m1_parts = [], []
    for s in range(NSEG):
        idx_s = adjT[:, s * SEG:(s + 1) * SEG].reshape(-1)
        g_s = _gather_feat(features, idx_s).reshape(S_HOP2, SEG, D_FEAT)
        h10_s, m1_s = pl.pallas_call(
            _fused0_body,
            grid=(SEG // NB,),
            in_specs=[
                pl.BlockSpec((S_HOP2, NB, D_FEAT), lambda i: (0, i, 0)),
                pl.BlockSpec((NB, D_FEAT),
                             lambda i, s=s: (SEG // NB * s + i, 0)),
                pl.BlockSpec((D_FEAT, HIDDEN), lambda i: (0, 0)),
                pl.BlockSpec((1, HIDDEN), lambda i: (0, 0)),
                pl.BlockSpec((HIDDEN, D_FEAT), lambda i: (0, 0)),
                pl.BlockSpec((D_FEAT, D_FEAT), lambda i: (0, 0)),
                pl.BlockSpec((2 * D_FEAT, HIDDEN), lambda i: (0, 0)),
                pl.BlockSpec((1, HIDDEN), lambda i: (0, 0)),
            ],
            out_specs=[
                pl.BlockSpec((NB, 2 * D_FEAT), lambda i: (i, 0)),
                pl.BlockSpec((NB, HIDDEN), lambda i: (i, 0)),
            ],
            out_shape=[
                jax.ShapeDtypeStruct((SEG, 2 * D_FEAT), jnp.float32),
                jax.ShapeDtypeStruct((SEG, HIDDEN), jnp.float32),
            ],
        )(g_s, feat_pad, mlp_W0, mlp_b0.reshape(1, HIDDEN),
          neigh_W0, self_W0, mlp_W1, mlp_b1.reshape(1, HIDDEN))
        h10_parts.append(h10_s)
        m1_parts.append(m1_s)
    h10_tab = jnp.concatenate(h10_parts)
    m1_tab = jnp.concatenate(m1_parts)

    batch = jnp.concatenate([batch1, batch2])
    adj128 = jnp.pad(adj, ((0, 0), (0, 128 - MAX_DEG)))
    adjb = _gather_adj(adj128, batch)
    idx3 = adjb[:, :S_HOP1].T.reshape(-1)
    mrows = _gather_m1(m1_tab, idx3).reshape(S_HOP1, 2 * BATCH, HIDDEN)
    h10b = _gather_h10(h10_tab, batch)

    out = pl.pallas_call(
        _final_body,
        grid=(2 * BATCH // RB,),
        in_specs=[
            pl.BlockSpec((S_HOP1, RB, HIDDEN), lambda i: (0, i, 0)),
            pl.BlockSpec((RB, 2 * D_FEAT), lambda i: (i, 0)),
            pl.BlockSpec((2 * D_FEAT, D_FEAT), lambda i: (0, 0)),
            pl.BlockSpec((HIDDEN, D_FEAT), lambda i: (0, 0)),
        ],
        out_specs=pl.BlockSpec((RB, 2 * D_FEAT), lambda i: (i, 0)),
        out_shape=jax.ShapeDtypeStruct((2 * BATCH, 2 * D_FEAT), jnp.float32),
    )(mrows, h10b, self_W1, neigh_W1)

    return (out[:BATCH], out[BATCH:])


# speedup vs baseline: 2.6133x; 1.8134x over previous
import functools

import jax
import jax.numpy as jnp
from jax import lax
from jax.experimental import pallas as pl
from jax.experimental.pallas import tpu as pltpu
from jax.experimental.pallas import tpu_sc as plsc

N_NODES = 10000
MAX_DEG = 32
D_FEAT = 128
HIDDEN = 512
S_HOP2 = 25
S_HOP1 = 10
BATCH = 512
NW = 32


def _make_sc_gather(V, D, B, dtype, chunk):
    per_w = B // NW
    assert B % NW == 0 and per_w % chunk == 0 and chunk % 8 == 0 and chunk <= 128
    n_chunks = per_w // chunk
    idx2d = n_chunks % 8 == 0
    pipelined = idx2d and n_chunks % 4 == 0
    mesh = plsc.VectorSubcoreMesh(core_axis_name="c", subcore_axis_name="s")

    @functools.partial(
        pl.kernel,
        mesh=mesh,
        out_type=jax.ShapeDtypeStruct((B, D), dtype),
        scratch_types=[
            pltpu.VMEM((n_chunks, chunk) if idx2d else (per_w,), jnp.int32),
        ] + [pltpu.VMEM((chunk, D), dtype) for _ in range(4 if pipelined else 1)]
          + [pltpu.SemaphoreType.DMA for _ in range(8 if pipelined else 1)],
    )
    def gk(table_hbm, idx_hbm, out_hbm, idx_v, *bufs_sems):
        wid = lax.axis_index("s") * 2 + lax.axis_index("c")
        base = wid * per_w

        def idx_at(c):
            return idx_v.at[c] if idx2d else idx_v.at[pl.ds(c * chunk, chunk)]

        if idx2d:
            pltpu.sync_copy(idx_hbm.at[pl.ds(wid * n_chunks, n_chunks)], idx_v)
        else:
            pltpu.sync_copy(idx_hbm.at[pl.ds(base, per_w)], idx_v)

        if not pipelined:
            buf_v, sem = bufs_sems

            def body(c, carry):
                pltpu.async_copy(table_hbm.at[idx_at(c)], buf_v, sem).wait()
                pltpu.sync_copy(buf_v, out_hbm.at[pl.ds(base + c * chunk, chunk)])
                return carry

            lax.fori_loop(0, n_chunks, body, 0)
            return

        bufs = bufs_sems[:4]
        gsems = bufs_sems[4:8]
        wsems = bufs_sems[8:12]
        outer = n_chunks // 4

        def gstart(c, b):
            pltpu.async_copy(table_hbm.at[idx_at(c)], bufs[b], gsems[b])

        def gwait(c, b):
            pltpu.make_async_copy(
                table_hbm.at[idx_at(c)], bufs[b], gsems[b]).wait()

        def wstart(c, b):
            pltpu.async_copy(bufs[b], out_hbm.at[pl.ds(base + c * chunk, chunk)],
                             wsems[b])

        def wwait(c, b):
            pltpu.make_async_copy(
                bufs[b], out_hbm.at[pl.ds(base + c * chunk, chunk)],
                wsems[b]).wait()

        gstart(0, 0)
        gstart(1, 1)

        def body(o, carry):
            c0 = 4 * o

            @pl.when(o > 0)
            def _():
                wwait(c0 - 2, 2)
                wwait(c0 - 1, 3)

            gstart(c0 + 2, 2)
            gstart(c0 + 3, 3)

            gwait(c0, 0)
            wstart(c0, 0)
            gwait(c0 + 1, 1)
            wstart(c0 + 1, 1)

            @pl.when(o + 1 < outer)
            def _():
                wwait(c0, 0)
                wwait(c0 + 1, 1)
                gstart(c0 + 4, 0)
                gstart(c0 + 5, 1)

            gwait(c0 + 2, 2)
            wstart(c0 + 2, 2)
            gwait(c0 + 3, 3)
            wstart(c0 + 3, 3)
            return carry

        lax.fori_loop(0, outer, body, 0)
        last = n_chunks - 4
        wwait(last, 0)
        wwait(last + 1, 1)
        wwait(last + 2, 2)
        wwait(last + 3, 3)

    def call(table, idx_flat):
        if idx2d:
            return gk(table, idx_flat.reshape(NW * n_chunks, chunk))
        return gk(table, idx_flat)

    return call


KPAD = 10240
NSEG = 4
SEG = KPAD // NSEG
B_SEG = S_HOP2 * SEG
_gather_feat = _make_sc_gather(N_NODES, D_FEAT, B_SEG, jnp.float32, 80)
_gather_adj = _make_sc_gather(N_NODES, 128, 2 * BATCH, jnp.int32, 32)
_gather_m1 = _make_sc_gather(N_NODES, HIDDEN, 2 * BATCH * S_HOP1, jnp.float32, 40)
_gather_h10 = _make_sc_gather(N_NODES, 2 * D_FEAT, 2 * BATCH, jnp.float32, 32)

NB = 256


def _fused0_body(g_ref, f_ref, w0_ref, b0_ref, nw0_ref, sw0_ref, w1_ref, b1_ref,
                 h10_ref, m1_ref):
    w0 = w0_ref[...]
    m10 = None
    for j in range(S_HOP2):
        d = jnp.dot(g_ref[j], w0, preferred_element_type=jnp.float32)
        m25 = d if j == 0 else jnp.maximum(m25, d)
        if j == S_HOP1 - 1:
            m10 = m25
    b0 = b0_ref[...]
    p25 = jax.nn.relu(m25 + b0)
    p10 = jax.nn.relu(m10 + b0)
    s0 = jnp.dot(f_ref[...], sw0_ref[...], preferred_element_type=jnp.float32)
    n25 = jnp.dot(p25, nw0_ref[...], preferred_element_type=jnp.float32)
    n10 = jnp.dot(p10, nw0_ref[...], preferred_element_type=jnp.float32)
    h25 = jax.nn.relu(jnp.concatenate([s0, n25], axis=1))
    h10_ref[...] = jax.nn.relu(jnp.concatenate([s0, n10], axis=1))
    m1_ref[...] = jax.nn.relu(
        jnp.dot(h25, w1_ref[...], preferred_element_type=jnp.float32) + b1_ref[...])


RB = 256


def _final_body(m_ref, h_ref, sw1_ref, nw1_ref, o_ref):
    pooled = m_ref[0]
    for j in range(1, S_HOP1):
        pooled = jnp.maximum(pooled, m_ref[j])
    a = jnp.dot(h_ref[...], sw1_ref[...], preferred_element_type=jnp.float32)
    b = jnp.dot(pooled, nw1_ref[...], preferred_element_type=jnp.float32)
    o = jnp.concatenate([a, b], axis=1)
    s = jnp.sum(o * o, axis=1, keepdims=True)
    o_ref[...] = o * lax.rsqrt(jnp.maximum(s, 1e-12))


def kernel(batch1, batch2, features, adj, mlp_W0, mlp_b0, neigh_W0, self_W0,
           mlp_W1, mlp_b1, neigh_W1, self_W1):
    pad_idx = jnp.broadcast_to(
        (jnp.arange(KPAD - N_NODES, dtype=jnp.int32) * 41) % N_NODES,
        (S_HOP2, KPAD - N_NODES))
    adjT = jnp.concatenate([adj[:, :S_HOP2].T, pad_idx], axis=1)
    feat_pad = jnp.pad(features, ((0, KPAD - N_NODES), (0, 0)))

    h10_parts, m1_parts = [], []
    for s in range(NSEG):
        idx_s = adjT[:, s * SEG:(s + 1) * SEG].reshape(-1)
        g_s = _gather_feat(features, idx_s).reshape(S_HOP2, SEG, D_FEAT)
        h10_s, m1_s = pl.pallas_call(
            _fused0_body,
            grid=(SEG // NB,),
            in_specs=[
                pl.BlockSpec((S_HOP2, NB, D_FEAT), lambda i: (0, i, 0)),
                pl.BlockSpec((NB, D_FEAT),
                             lambda i, s=s: (SEG // NB * s + i, 0)),
                pl.BlockSpec((D_FEAT, HIDDEN), lambda i: (0, 0)),
                pl.BlockSpec((1, HIDDEN), lambda i: (0, 0)),
                pl.BlockSpec((HIDDEN, D_FEAT), lambda i: (0, 0)),
                pl.BlockSpec((D_FEAT, D_FEAT), lambda i: (0, 0)),
                pl.BlockSpec((2 * D_FEAT, HIDDEN), lambda i: (0, 0)),
                pl.BlockSpec((1, HIDDEN), lambda i: (0, 0)),
            ],
            out_specs=[
                pl.BlockSpec((NB, 2 * D_FEAT), lambda i: (i, 0)),
                pl.BlockSpec((NB, HIDDEN), lambda i: (i, 0)),
            ],
            out_shape=[
                jax.ShapeDtypeStruct((SEG, 2 * D_FEAT), jnp.float32),
                jax.ShapeDtypeStruct((SEG, HIDDEN), jnp.float32),
            ],
        )(g_s, feat_pad, mlp_W0, mlp_b0.reshape(1, HIDDEN),
          neigh_W0, self_W0, mlp_W1, mlp_b1.reshape(1, HIDDEN))
        h10_parts.append(h10_s)
        m1_parts.append(m1_s)
    h10_tab = jnp.concatenate(h10_parts)
    m1_tab = jnp.concatenate(m1_parts)

    batch = jnp.concatenate([batch1, batch2])
    adj128 = jnp.pad(adj, ((0, 0), (0, 128 - MAX_DEG)))
    adjb = _gather_adj(adj128, batch)
    idx3 = adjb[:, :S_HOP1].T.reshape(-1)
    mrows = _gather_m1(m1_tab, idx3).reshape(S_HOP1, 2 * BATCH, HIDDEN)
    h10b = _gather_h10(h10_tab, batch)

    out = pl.pallas_call(
        _final_body,
        grid=(2 * BATCH // RB,),
        in_specs=[
            pl.BlockSpec((S_HOP1, RB, HIDDEN), lambda i: (0, i, 0)),
            pl.BlockSpec((RB, 2 * D_FEAT), lambda i: (i, 0)),
            pl.BlockSpec((2 * D_FEAT, D_FEAT), lambda i: (0, 0)),
            pl.BlockSpec((HIDDEN, D_FEAT), lambda i: (0, 0)),
        ],
        out_specs=pl.BlockSpec((RB, 2 * D_FEAT), lambda i: (i, 0)),
        out_shape=jax.ShapeDtypeStruct((2 * BATCH, 2 * D_FEAT), jnp.float32),
    )(mrows, h10b, self_W1, neigh_W1)

    return (out[:BATCH], out[BATCH:])


# speedup vs baseline: 3.2192x; 1.2318x over previous
import functools

import jax
import jax.numpy as jnp
from jax import lax
from jax.experimental import pallas as pl
from jax.experimental.pallas import tpu as pltpu
from jax.experimental.pallas import tpu_sc as plsc

N_NODES = 10000
MAX_DEG = 32
D_FEAT = 128
HIDDEN = 512
S_HOP2 = 25
S_HOP1 = 10
BATCH = 512
NW = 32


def _make_sc_gather(V, D, B, dtype, chunk):
    per_w = B // NW
    assert B % NW == 0 and per_w % chunk == 0 and chunk % 8 == 0 and chunk <= 128
    n_chunks = per_w // chunk
    idx2d = n_chunks % 8 == 0
    pipelined = idx2d and n_chunks % 4 == 0
    mesh = plsc.VectorSubcoreMesh(core_axis_name="c", subcore_axis_name="s")

    @functools.partial(
        pl.kernel,
        mesh=mesh,
        out_type=jax.ShapeDtypeStruct((B, D), dtype),
        scratch_types=[
            pltpu.VMEM((n_chunks, chunk) if idx2d else (per_w,), jnp.int32),
        ] + [pltpu.VMEM((chunk, D), dtype) for _ in range(4 if pipelined else 1)]
          + [pltpu.SemaphoreType.DMA for _ in range(8 if pipelined else 1)],
    )
    def gk(table_hbm, idx_hbm, out_hbm, idx_v, *bufs_sems):
        wid = lax.axis_index("s") * 2 + lax.axis_index("c")
        base = wid * per_w

        def idx_at(c):
            return idx_v.at[c] if idx2d else idx_v.at[pl.ds(c * chunk, chunk)]

        if idx2d:
            pltpu.sync_copy(idx_hbm.at[pl.ds(wid * n_chunks, n_chunks)], idx_v)
        else:
            pltpu.sync_copy(idx_hbm.at[pl.ds(base, per_w)], idx_v)

        if not pipelined:
            buf_v, sem = bufs_sems

            def body(c, carry):
                pltpu.async_copy(table_hbm.at[idx_at(c)], buf_v, sem).wait()
                pltpu.sync_copy(buf_v, out_hbm.at[pl.ds(base + c * chunk, chunk)])
                return carry

            lax.fori_loop(0, n_chunks, body, 0)
            return

        bufs = bufs_sems[:4]
        gsems = bufs_sems[4:8]
        wsems = bufs_sems[8:12]
        outer = n_chunks // 4

        def gstart(c, b):
            pltpu.async_copy(table_hbm.at[idx_at(c)], bufs[b], gsems[b])

        def gwait(c, b):
            pltpu.make_async_copy(
                table_hbm.at[idx_at(c)], bufs[b], gsems[b]).wait()

        def wstart(c, b):
            pltpu.async_copy(bufs[b], out_hbm.at[pl.ds(base + c * chunk, chunk)],
                             wsems[b])

        def wwait(c, b):
            pltpu.make_async_copy(
                bufs[b], out_hbm.at[pl.ds(base + c * chunk, chunk)],
                wsems[b]).wait()

        gstart(0, 0)
        gstart(1, 1)

        def body(o, carry):
            c0 = 4 * o

            @pl.when(o > 0)
            def _():
                wwait(c0 - 2, 2)
                wwait(c0 - 1, 3)

            gstart(c0 + 2, 2)
            gstart(c0 + 3, 3)

            gwait(c0, 0)
            wstart(c0, 0)
            gwait(c0 + 1, 1)
            wstart(c0 + 1, 1)

            @pl.when(o + 1 < outer)
            def _():
                wwait(c0, 0)
                wwait(c0 + 1, 1)
                gstart(c0 + 4, 0)
                gstart(c0 + 5, 1)

            gwait(c0 + 2, 2)
            wstart(c0 + 2, 2)
            gwait(c0 + 3, 3)
            wstart(c0 + 3, 3)
            return carry

        lax.fori_loop(0, outer, body, 0)
        last = n_chunks - 4
        wwait(last, 0)
        wwait(last + 1, 1)
        wwait(last + 2, 2)
        wwait(last + 3, 3)

    def call(table, idx_flat):
        if idx2d:
            return gk(table, idx_flat.reshape(NW * n_chunks, chunk))
        return gk(table, idx_flat)

    return call


KPAD = 10240
NSEG = 5
SEG = KPAD // NSEG
B_SEG = S_HOP2 * SEG
_gather_feat = _make_sc_gather(N_NODES, D_FEAT, B_SEG, jnp.float32, 40)
_gather_adj = _make_sc_gather(N_NODES, 128, 2 * BATCH, jnp.int32, 32)
_gather_m1 = _make_sc_gather(N_NODES, HIDDEN, 2 * BATCH * S_HOP1, jnp.float32, 40)
_gather_h10 = _make_sc_gather(N_NODES, 2 * D_FEAT, 2 * BATCH, jnp.float32, 32)

NB = 256


def _fused0_body(g_ref, f_ref, w0_ref, b0_ref, nw0_ref, sw0_ref, w1_ref, b1_ref,
                 h10_ref, m1_ref):
    w0 = w0_ref[...]
    m10 = None
    for j in range(S_HOP2):
        d = jnp.dot(g_ref[j], w0, preferred_element_type=jnp.float32)
        m25 = d if j == 0 else jnp.maximum(m25, d)
        if j == S_HOP1 - 1:
            m10 = m25
    b0 = b0_ref[...]
    p25 = jax.nn.relu(m25 + b0)
    p10 = jax.nn.relu(m10 + b0)
    s0 = jnp.dot(f_ref[...], sw0_ref[...], preferred_element_type=jnp.float32)
    n25 = jnp.dot(p25, nw0_ref[...], preferred_element_type=jnp.float32)
    n10 = jnp.dot(p10, nw0_ref[...], preferred_element_type=jnp.float32)
    h25 = jax.nn.relu(jnp.concatenate([s0, n25], axis=1))
    h10_ref[...] = jax.nn.relu(jnp.concatenate([s0, n10], axis=1))
    m1_ref[...] = jax.nn.relu(
        jnp.dot(h25, w1_ref[...], preferred_element_type=jnp.float32) + b1_ref[...])


RB = 256


def _final_body(m_ref, h_ref, sw1_ref, nw1_ref, o_ref):
    pooled = m_ref[0]
    for j in range(1, S_HOP1):
        pooled = jnp.maximum(pooled, m_ref[j])
    a = jnp.dot(h_ref[...], sw1_ref[...], preferred_element_type=jnp.float32)
    b = jnp.dot(pooled, nw1_ref[...], preferred_element_type=jnp.float32)
    o = jnp.concatenate([a, b], axis=1)
    s = jnp.sum(o * o, axis=1, keepdims=True)
    o_ref[...] = o * lax.rsqrt(jnp.maximum(s, 1e-12))


def kernel(batch1, batch2, features, adj, mlp_W0, mlp_b0, neigh_W0, self_W0,
           mlp_W1, mlp_b1, neigh_W1, self_W1):
    pad_idx = jnp.broadcast_to(
        (jnp.arange(KPAD - N_NODES, dtype=jnp.int32) * 41) % N_NODES,
        (S_HOP2, KPAD - N_NODES))
    adjT = jnp.concatenate([adj[:, :S_HOP2].T, pad_idx], axis=1)
    feat_pad = jnp.pad(features, ((0, KPAD - N_NODES), (0, 0)))

    h10_parts, m1_parts = [], []
    for s in range(NSEG):
        idx_s = adjT[:, s * SEG:(s + 1) * SEG].reshape(-1)
        g_s = _gather_feat(features, idx_s).reshape(S_HOP2, SEG, D_FEAT)
        h10_s, m1_s = pl.pallas_call(
            _fused0_body,
            grid=(SEG // NB,),
            in_specs=[
                pl.BlockSpec((S_HOP2, NB, D_FEAT), lambda i: (0, i, 0)),
                pl.BlockSpec((NB, D_FEAT),
                             lambda i, s=s: (SEG // NB * s + i, 0)),
                pl.BlockSpec((D_FEAT, HIDDEN), lambda i: (0, 0)),
                pl.BlockSpec((1, HIDDEN), lambda i: (0, 0)),
                pl.BlockSpec((HIDDEN, D_FEAT), lambda i: (0, 0)),
                pl.BlockSpec((D_FEAT, D_FEAT), lambda i: (0, 0)),
                pl.BlockSpec((2 * D_FEAT, HIDDEN), lambda i: (0, 0)),
                pl.BlockSpec((1, HIDDEN), lambda i: (0, 0)),
            ],
            out_specs=[
                pl.BlockSpec((NB, 2 * D_FEAT), lambda i: (i, 0)),
                pl.BlockSpec((NB, HIDDEN), lambda i: (i, 0)),
            ],
            out_shape=[
                jax.ShapeDtypeStruct((SEG, 2 * D_FEAT), jnp.float32),
                jax.ShapeDtypeStruct((SEG, HIDDEN), jnp.float32),
            ],
        )(g_s, feat_pad, mlp_W0, mlp_b0.reshape(1, HIDDEN),
          neigh_W0, self_W0, mlp_W1, mlp_b1.reshape(1, HIDDEN))
        h10_parts.append(h10_s)
        m1_parts.append(m1_s)
    h10_tab = jnp.concatenate(h10_parts)
    m1_tab = jnp.concatenate(m1_parts)

    batch = jnp.concatenate([batch1, batch2])
    adj128 = jnp.pad(adj, ((0, 0), (0, 128 - MAX_DEG)))
    adjb = _gather_adj(adj128, batch)
    idx3 = adjb[:, :S_HOP1].T.reshape(-1)
    mrows = _gather_m1(m1_tab, idx3).reshape(S_HOP1, 2 * BATCH, HIDDEN)
    h10b = _gather_h10(h10_tab, batch)

    out = pl.pallas_call(
        _final_body,
        grid=(2 * BATCH // RB,),
        in_specs=[
            pl.BlockSpec((S_HOP1, RB, HIDDEN), lambda i: (0, i, 0)),
            pl.BlockSpec((RB, 2 * D_FEAT), lambda i: (i, 0)),
            pl.BlockSpec((2 * D_FEAT, D_FEAT), lambda i: (0, 0)),
            pl.BlockSpec((HIDDEN, D_FEAT), lambda i: (0, 0)),
        ],
        out_specs=pl.BlockSpec((RB, 2 * D_FEAT), lambda i: (i, 0)),
        out_shape=jax.ShapeDtypeStruct((2 * BATCH, 2 * D_FEAT), jnp.float32),
    )(mrows, h10b, self_W1, neigh_W1)

    return (out[:BATCH], out[BATCH:])


# speedup vs baseline: 3.4004x; 1.0563x over previous
import functools

import jax
import jax.numpy as jnp
from jax import lax
from jax.experimental import pallas as pl
from jax.experimental.pallas import tpu as pltpu
from jax.experimental.pallas import tpu_sc as plsc

N_NODES = 10000
MAX_DEG = 32
D_FEAT = 128
HIDDEN = 512
S_HOP2 = 25
S_HOP1 = 10
BATCH = 512
NW = 32


def _make_sc_gather(V, D, B, dtype, chunk):
    per_w = B // NW
    assert B % NW == 0 and per_w % chunk == 0 and chunk % 8 == 0 and chunk <= 128
    n_chunks = per_w // chunk
    idx2d = n_chunks % 8 == 0
    pipelined = idx2d and n_chunks % 4 == 0
    mesh = plsc.VectorSubcoreMesh(core_axis_name="c", subcore_axis_name="s")

    @functools.partial(
        pl.kernel,
        mesh=mesh,
        out_type=jax.ShapeDtypeStruct((B, D), dtype),
        scratch_types=[
            pltpu.VMEM((n_chunks, chunk) if idx2d else (per_w,), jnp.int32),
        ] + [pltpu.VMEM((chunk, D), dtype) for _ in range(4 if pipelined else 1)]
          + [pltpu.SemaphoreType.DMA for _ in range(8 if pipelined else 1)],
    )
    def gk(table_hbm, idx_hbm, out_hbm, idx_v, *bufs_sems):
        wid = lax.axis_index("s") * 2 + lax.axis_index("c")
        base = wid * per_w

        def idx_at(c):
            return idx_v.at[c] if idx2d else idx_v.at[pl.ds(c * chunk, chunk)]

        if idx2d:
            pltpu.sync_copy(idx_hbm.at[pl.ds(wid * n_chunks, n_chunks)], idx_v)
        else:
            pltpu.sync_copy(idx_hbm.at[pl.ds(base, per_w)], idx_v)

        if not pipelined:
            buf_v, sem = bufs_sems

            def body(c, carry):
                pltpu.async_copy(table_hbm.at[idx_at(c)], buf_v, sem).wait()
                pltpu.sync_copy(buf_v, out_hbm.at[pl.ds(base + c * chunk, chunk)])
                return carry

            lax.fori_loop(0, n_chunks, body, 0)
            return

        bufs = bufs_sems[:4]
        gsems = bufs_sems[4:8]
        wsems = bufs_sems[8:12]
        outer = n_chunks // 4

        def gstart(c, b):
            pltpu.async_copy(table_hbm.at[idx_at(c)], bufs[b], gsems[b])

        def gwait(c, b):
            pltpu.make_async_copy(
                table_hbm.at[idx_at(c)], bufs[b], gsems[b]).wait()

        def wstart(c, b):
            pltpu.async_copy(bufs[b], out_hbm.at[pl.ds(base + c * chunk, chunk)],
                             wsems[b])

        def wwait(c, b):
            pltpu.make_async_copy(
                bufs[b], out_hbm.at[pl.ds(base + c * chunk, chunk)],
                wsems[b]).wait()

        gstart(0, 0)
        gstart(1, 1)

        def body(o, carry):
            c0 = 4 * o

            @pl.when(o > 0)
            def _():
                wwait(c0 - 2, 2)
                wwait(c0 - 1, 3)

            gstart(c0 + 2, 2)
            gstart(c0 + 3, 3)

            gwait(c0, 0)
            wstart(c0, 0)
            gwait(c0 + 1, 1)
            wstart(c0 + 1, 1)

            @pl.when(o + 1 < outer)
            def _():
                wwait(c0, 0)
                wwait(c0 + 1, 1)
                gstart(c0 + 4, 0)
                gstart(c0 + 5, 1)

            gwait(c0 + 2, 2)
            wstart(c0 + 2, 2)
            gwait(c0 + 3, 3)
            wstart(c0 + 3, 3)
            return carry

        lax.fori_loop(0, outer, body, 0)
        last = n_chunks - 4
        wwait(last, 0)
        wwait(last + 1, 1)
        wwait(last + 2, 2)
        wwait(last + 3, 3)

    def call(table, idx_flat):
        if idx2d:
            return gk(table, idx_flat.reshape(NW * n_chunks, chunk))
        return gk(table, idx_flat)

    return call


KPAD = 10240
NSEG = 5
SEG = KPAD // NSEG
B_SEG = S_HOP2 * SEG
_gather_feat = _make_sc_gather(N_NODES, D_FEAT, B_SEG, jnp.float32, 40)
_gather_adj = _make_sc_gather(N_NODES, 128, 2 * BATCH, jnp.int32, 32)
_gather_m1 = _make_sc_gather(N_NODES, HIDDEN, 2 * BATCH * S_HOP1, jnp.float32, 40)
_gather_h10 = _make_sc_gather(N_NODES, 2 * D_FEAT, 2 * BATCH, jnp.float32, 32)

NB = 256


def _fused0_body(g_ref, f_ref, w0_ref, b0_ref, nw0_ref, sw0_ref, w1_ref, b1_ref,
                 h10_ref, m1_ref):
    w0 = w0_ref[...]
    m10 = None
    for j in range(S_HOP2):
        d = jnp.dot(g_ref[j], w0, preferred_element_type=jnp.float32)
        m25 = d if j == 0 else jnp.maximum(m25, d)
        if j == S_HOP1 - 1:
            m10 = m25
    b0 = b0_ref[...]
    p25 = jax.nn.relu(m25 + b0)
    p10 = jax.nn.relu(m10 + b0)
    s0 = jnp.dot(f_ref[...], sw0_ref[...], preferred_element_type=jnp.float32)
    n25 = jnp.dot(p25, nw0_ref[...], preferred_element_type=jnp.float32)
    n10 = jnp.dot(p10, nw0_ref[...], preferred_element_type=jnp.float32)
    h25 = jax.nn.relu(jnp.concatenate([s0, n25], axis=1))
    h10_ref[...] = jax.nn.relu(jnp.concatenate([s0, n10], axis=1))
    m1_ref[...] = jax.nn.relu(
        jnp.dot(h25, w1_ref[...], preferred_element_type=jnp.float32) + b1_ref[...])


def _fused0_alias_body(g, f, w0, b0, nw0, sw0, w1, b1, h10_in, m1_in,
                       h10_out, m1_out):
    del h10_in, m1_in
    _fused0_body(g, f, w0, b0, nw0, sw0, w1, b1, h10_out, m1_out)


RB = 256


def _final_body(m_ref, h_ref, sw1_ref, nw1_ref, o_ref):
    pooled = m_ref[0]
    for j in range(1, S_HOP1):
        pooled = jnp.maximum(pooled, m_ref[j])
    a = jnp.dot(h_ref[...], sw1_ref[...], preferred_element_type=jnp.float32)
    b = jnp.dot(pooled, nw1_ref[...], preferred_element_type=jnp.float32)
    o = jnp.concatenate([a, b], axis=1)
    s = jnp.sum(o * o, axis=1, keepdims=True)
    o_ref[...] = o * lax.rsqrt(jnp.maximum(s, 1e-12))


def kernel(batch1, batch2, features, adj, mlp_W0, mlp_b0, neigh_W0, self_W0,
           mlp_W1, mlp_b1, neigh_W1, self_W1):
    pad_idx = jnp.broadcast_to(
        (jnp.arange(KPAD - N_NODES, dtype=jnp.int32) * 41) % N_NODES,
        (S_HOP2, KPAD - N_NODES))
    adjT = jnp.concatenate([adj[:, :S_HOP2].T, pad_idx], axis=1)
    feat_pad = jnp.pad(features, ((0, KPAD - N_NODES), (0, 0)))

    h10_tab = m1_tab = None
    for s in range(NSEG):
        idx_s = adjT[:, s * SEG:(s + 1) * SEG].reshape(-1)
        g_s = _gather_feat(features, idx_s).reshape(S_HOP2, SEG, D_FEAT)
        in_specs = [
            pl.BlockSpec((S_HOP2, NB, D_FEAT), lambda i: (0, i, 0)),
            pl.BlockSpec((NB, D_FEAT),
                         lambda i, s=s: (SEG // NB * s + i, 0)),
            pl.BlockSpec((D_FEAT, HIDDEN), lambda i: (0, 0)),
            pl.BlockSpec((1, HIDDEN), lambda i: (0, 0)),
            pl.BlockSpec((HIDDEN, D_FEAT), lambda i: (0, 0)),
            pl.BlockSpec((D_FEAT, D_FEAT), lambda i: (0, 0)),
            pl.BlockSpec((2 * D_FEAT, HIDDEN), lambda i: (0, 0)),
            pl.BlockSpec((1, HIDDEN), lambda i: (0, 0)),
        ]
        out_specs = [
            pl.BlockSpec((NB, 2 * D_FEAT), lambda i, s=s: (SEG // NB * s + i, 0)),
            pl.BlockSpec((NB, HIDDEN), lambda i, s=s: (SEG // NB * s + i, 0)),
        ]
        out_shape = [
            jax.ShapeDtypeStruct((KPAD, 2 * D_FEAT), jnp.float32),
            jax.ShapeDtypeStruct((KPAD, HIDDEN), jnp.float32),
        ]
        args = (g_s, feat_pad, mlp_W0, mlp_b0.reshape(1, HIDDEN),
                neigh_W0, self_W0, mlp_W1, mlp_b1.reshape(1, HIDDEN))
        if s == 0:
            h10_tab, m1_tab = pl.pallas_call(
                _fused0_body, grid=(SEG // NB,), in_specs=in_specs,
                out_specs=out_specs, out_shape=out_shape)(*args)
        else:
            h10_tab, m1_tab = pl.pallas_call(
                _fused0_alias_body, grid=(SEG // NB,),
                in_specs=in_specs + [
                    pl.BlockSpec((8, 2 * D_FEAT), lambda i: (0, 0)),
                    pl.BlockSpec((8, HIDDEN), lambda i: (0, 0)),
                ],
                out_specs=out_specs, out_shape=out_shape,
                input_output_aliases={8: 0, 9: 1},
            )(*args, h10_tab, m1_tab)

    batch = jnp.concatenate([batch1, batch2])
    adj128 = jnp.pad(adj, ((0, 0), (0, 128 - MAX_DEG)))
    adjb = _gather_adj(adj128, batch)
    idx3 = adjb[:, :S_HOP1].T.reshape(-1)
    mrows = _gather_m1(m1_tab, idx3).reshape(S_HOP1, 2 * BATCH, HIDDEN)
    h10b = _gather_h10(h10_tab, batch)

    out = pl.pallas_call(
        _final_body,
        grid=(2 * BATCH // RB,),
        in_specs=[
            pl.BlockSpec((S_HOP1, RB, HIDDEN), lambda i: (0, i, 0)),
            pl.BlockSpec((RB, 2 * D_FEAT), lambda i: (i, 0)),
            pl.BlockSpec((2 * D_FEAT, D_FEAT), lambda i: (0, 0)),
            pl.BlockSpec((HIDDEN, D_FEAT), lambda i: (0, 0)),
        ],
        out_specs=pl.BlockSpec((RB, 2 * D_FEAT), lambda i: (i, 0)),
        out_shape=jax.ShapeDtypeStruct((2 * BATCH, 2 * D_FEAT), jnp.float32),
    )(mrows, h10b, self_W1, neigh_W1)

    return (out[:BATCH], out[BATCH:])
